# baseline (speedup 1.0000x reference)
import functools

import jax
import jax.numpy as jnp
from jax.experimental import pallas as pl
from jax.experimental.pallas import tpu as pltpu
from jax.experimental.pallas import tpu_sc as plsc

F32 = jnp.float32
BF16 = jnp.bfloat16

_SC_CORES = 2
_SC_SUBCORES = 16
_SC_WORKERS = _SC_CORES * _SC_SUBCORES


def _sc_gather(table, idx):
    n = idx.shape[0]
    d = table.shape[1]
    bpw = n // _SC_WORKERS
    mesh = plsc.VectorSubcoreMesh(core_axis_name="c", subcore_axis_name="s")

    @functools.partial(
        pl.kernel,
        mesh=mesh,
        out_type=jax.ShapeDtypeStruct((n, d), table.dtype),
        scratch_types=[
            pltpu.VMEM((bpw,), jnp.int32),
            pltpu.VMEM((bpw, d), table.dtype),
            pltpu.SemaphoreType.DMA,
        ],
    )
    def gather_kernel(table_hbm, idx_hbm, out_hbm, idx_v, rows_v, sem):
        wid = jax.lax.axis_index("s") * _SC_CORES + jax.lax.axis_index("c")
        base = wid * bpw
        pltpu.sync_copy(idx_hbm.at[pl.ds(base, bpw)], idx_v)
        pltpu.async_copy(table_hbm.at[idx_v], rows_v, sem).wait()
        pltpu.sync_copy(rows_v, out_hbm.at[pl.ds(base, bpw)])

    return gather_kernel(table, idx)


def _sigmoid(x):
    return jax.nn.sigmoid(x)


def _vattn(memT, q, mask):
    prod = memT * q.astype(BF16)[:, None, :]
    s = jnp.sum(prod, axis=-1, dtype=F32) + mask
    s = s - jnp.max(s, axis=-1, keepdims=True)
    e = jnp.exp(s)
    a = e / jnp.sum(e, axis=-1, keepdims=True)
    ctx = jnp.sum(a.astype(BF16)[:, :, None] * memT, axis=1, dtype=F32)
    return a, ctx


def _enc_kernel(x_ref, wx_ref, wh_ref, b_ref, memT_ref, hT_ref, cT_ref,
                h_s, c_s, xg_s, *, chunk, nB, Hh):
    ci = pl.program_id(0)

    @pl.when(ci == 0)
    def _():
        h_s[:] = jnp.zeros_like(h_s)
        c_s[:] = jnp.zeros_like(c_s)

    xg_s[:] = jnp.dot(x_ref[:], wx_ref[:], preferred_element_type=F32) + b_ref[:]

    def step(t, _):
        hb = h_s[:].astype(BF16)
        xg = xg_s[pl.ds(t * nB, nB)]
        g3 = xg[:, :3 * Hh] + jnp.dot(hb, wh_ref[:, :3 * Hh],
                                      preferred_element_type=F32)
        i_ = _sigmoid(g3[:, :Hh])
        f_ = _sigmoid(g3[:, Hh:2 * Hh])
        gg = jnp.tanh(g3[:, 2 * Hh:])
        go = xg[:, 3 * Hh:] + jnp.dot(hb, wh_ref[:, 3 * Hh:],
                                      preferred_element_type=F32)
        c2 = f_ * c_s[:] + i_ * gg
        h2 = _sigmoid(go) * jnp.tanh(c2)
        memT_ref[:, t, :] = h2.astype(BF16)
        h_s[:] = h2
        c_s[:] = c2
        return 0

    jax.lax.fori_loop(0, chunk, step, 0, unroll=True)
    hT_ref[:] = h_s[:]
    cT_ref[:] = c_s[:]


def _lstm_encoder(x2d, Wx, Wh, b, nsteps, chunk, nB):
    D = x2d.shape[1]
    Hh = Wh.shape[0]
    H4 = Wx.shape[1]
    nchunks = nsteps // chunk
    kern = functools.partial(_enc_kernel, chunk=chunk, nB=nB, Hh=Hh)
    memT, hT, cT = pl.pallas_call(
        kern,
        grid=(nchunks,),
        in_specs=[
            pl.BlockSpec((chunk * nB, D), lambda c: (c, 0)),
            pl.BlockSpec((D, H4), lambda c: (0, 0)),
            pl.BlockSpec((Hh, H4), lambda c: (0, 0)),
            pl.BlockSpec((1, H4), lambda c: (0, 0)),
        ],
        out_specs=[
            pl.BlockSpec((nB, chunk, Hh), lambda c: (0, c, 0)),
            pl.BlockSpec((nB, Hh), lambda c: (0, 0)),
            pl.BlockSpec((nB, Hh), lambda c: (0, 0)),
        ],
        out_shape=[
            jax.ShapeDtypeStruct((nB, nsteps, Hh), BF16),
            jax.ShapeDtypeStruct((nB, Hh), F32),
            jax.ShapeDtypeStruct((nB, Hh), F32),
        ],
        scratch_shapes=[
            pltpu.VMEM((nB, Hh), F32),
            pltpu.VMEM((nB, Hh), F32),
            pltpu.VMEM((chunk * nB, H4), F32),
        ],
    )(x2d, Wx, Wh, b)
    return memT, hT, cT


def _head_kernel(memT_ref, infT_ref, wa_ref, wi_ref, wg_ref, bg_ref,
                 wc4_ref, bc_ref, ms_ref, mi_ref,
                 gas_ref, gai_ref, cst_ref, *, Hh):
    pos = infT_ref[:, 0, :].astype(F32)
    qs = jnp.dot(pos, wa_ref[:], preferred_element_type=F32)
    qi = jnp.dot(pos, wi_ref[:], preferred_element_type=F32)

    a_s, ctx_s = _vattn(memT_ref[:], qs, ms_ref[:])
    a_i, ctx_i = _vattn(infT_ref[:], qi, mi_ref[:])

    cat = jnp.concatenate([ctx_s, ctx_i], axis=1)
    gate = _sigmoid(jnp.dot(cat, wg_ref[:], preferred_element_type=F32)
                    + bg_ref[:])
    g_mem = gate * ctx_s + (1.0 - gate) * ctx_i

    gas_ref[:] = a_s
    gai_ref[:] = a_i
    cst_ref[:] = jnp.dot(g_mem, wc4_ref[:], preferred_element_type=F32) + bc_ref[:]


def _gated_head(memT, infT, Wa, Wi, Wg, bg, Wc4, bc, mask_s, mask_i):
    nB, Ls, Hh = memT.shape
    Li = infT.shape[1]
    kern = functools.partial(_head_kernel, Hh=Hh)
    gas, gai, cst = pl.pallas_call(
        kern,
        out_shape=[
            jax.ShapeDtypeStruct((nB, Ls), F32),
            jax.ShapeDtypeStruct((nB, Li), F32),
            jax.ShapeDtypeStruct((nB, Hh), F32),
        ],
    )(memT, infT, Wa, Wi, Wg, bg, Wc4, bc, mask_s, mask_i)
    return gas, gai, cst


def _dec_kernel(x_ref, wx1_ref, b_ref, wx2_ref, wh_ref, wai_ref, wc3_ref,
                cst_ref, h0_ref, c0_ref, memT_ref, infT_ref, ms_ref, mi_ref,
                out_ref, as_ref, ai_ref,
                hwh_s, c_s, fd_s, xg_s, *, chunk, nB, Hh):
    ci = pl.program_id(0)

    @pl.when(ci == 0)
    def _():
        hwh_s[:] = jnp.dot(h0_ref[:].astype(BF16), wh_ref[:],
                           preferred_element_type=F32)
        c_s[:] = c0_ref[:]
        fd_s[:] = jnp.zeros_like(fd_s)

    xg_s[:] = jnp.dot(x_ref[:], wx1_ref[:], preferred_element_type=F32) + b_ref[:]

    memT = memT_ref[:]
    infT = infT_ref[:]
    ms = ms_ref[:]
    mi = mi_ref[:]
    cst = cst_ref[:]

    Ls = memT.shape[1]
    Li = infT.shape[1]
    mem2d = memT.reshape(nB * Ls, Hh)
    inf2d = infT.reshape(nB * Li, Hh)
    eye3 = (jax.lax.broadcasted_iota(jnp.int32, (nB, 1, nB), 0)
            == jax.lax.broadcasted_iota(jnp.int32, (nB, 1, nB), 2)
            ).astype(F32)
    sel_s = (jax.lax.broadcasted_iota(jnp.int32, (nB, nB * Ls), 1) // Ls
             == jax.lax.broadcasted_iota(jnp.int32, (nB, nB * Ls), 0)
             ).astype(F32)
    sel_i = (jax.lax.broadcasted_iota(jnp.int32, (nB, nB * Li), 1) // Li
             == jax.lax.broadcasted_iota(jnp.int32, (nB, nB * Li), 0)
             ).astype(F32)

    def _mattn(m2d, L, q, mask, sel):
        S = jax.lax.dot_general(m2d, q.astype(BF16),
                                (((1,), (1,)), ((), ())),
                                preferred_element_type=F32)
        s = jnp.sum(S.reshape(nB, L, nB) * eye3, axis=-1) + mask
        s = s - jnp.max(s, axis=-1, keepdims=True)
        e = jnp.exp(s)
        a = e / jnp.sum(e, axis=-1, keepdims=True)
        A = (jnp.concatenate([a] * nB, axis=1) * sel).astype(BF16)
        ctx = jnp.dot(A, m2d, preferred_element_type=F32)
        return a, ctx

    def step(t, _):
        g = xg_s[pl.ds(t * nB, nB)] + hwh_s[:] + jnp.dot(
            fd_s[:].astype(BF16), wx2_ref[:], preferred_element_type=F32)
        i_ = _sigmoid(g[:, :Hh])
        f_ = _sigmoid(g[:, Hh:2 * Hh])
        gg = jnp.tanh(g[:, 2 * Hh:3 * Hh])
        o_ = _sigmoid(g[:, 3 * Hh:])
        c2 = f_ * c_s[:] + i_ * gg
        h2 = o_ * jnp.tanh(c2)
        h2b = h2.astype(BF16)

        q = jnp.dot(h2b, wai_ref[:], preferred_element_type=F32)
        qs = q[:, :Hh]
        qi = q[:, Hh:]

        hwh_s[:] = jnp.dot(h2b, wh_ref[:], preferred_element_type=F32)

        a_s, ctx_s = _mattn(mem2d, Ls, qs, ms, sel_s)
        a_i, ctx_i = _mattn(inf2d, Li, qi, mi, sel_i)

        hcc = jnp.concatenate([h2, ctx_s, ctx_i], axis=1).astype(BF16)
        out = jnp.tanh(jnp.dot(hcc, wc3_ref[:], preferred_element_type=F32)
                       + cst)

        out_ref[t] = out
        as_ref[t] = a_s
        ai_ref[t] = a_i
        c_s[:] = c2
        fd_s[:] = out
        return 0

    jax.lax.fori_loop(0, chunk, step, 0, unroll=True)


def _decoder(x2d, Wx1, b, Wx2, Wh, Wai, Wc3, cst, h0, c0, memT, infT,
             mask_s, mask_i, nsteps, chunk, nB):
    D = x2d.shape[1]
    Hh = h0.shape[1]
    H4 = Wx1.shape[1]
    Ls = memT.shape[1]
    Li = infT.shape[1]
    nchunks = nsteps // chunk
    kern = functools.partial(_dec_kernel, chunk=chunk, nB=nB, Hh=Hh)
    dec_out, a_std, a_inf = pl.pallas_call(
        kern,
        grid=(nchunks,),
        in_specs=[
            pl.BlockSpec((chunk * nB, D), lambda c: (c, 0)),
            pl.BlockSpec((D, H4), lambda c: (0, 0)),
            pl.BlockSpec((1, H4), lambda c: (0, 0)),
            pl.BlockSpec((Hh, H4), lambda c: (0, 0)),
            pl.BlockSpec((Hh, H4), lambda c: (0, 0)),
            pl.BlockSpec((Hh, 2 * Hh), lambda c: (0, 0)),
            pl.BlockSpec((3 * Hh, Hh), lambda c: (0, 0)),
            pl.BlockSpec((nB, Hh), lambda c: (0, 0)),
            pl.BlockSpec((nB, Hh), lambda c: (0, 0)),
            pl.BlockSpec((nB, Hh), lambda c: (0, 0)),
            pl.BlockSpec((nB, Ls, Hh), lambda c: (0, 0, 0)),
            pl.BlockSpec((nB, Li, Hh), lambda c: (0, 0, 0)),
            pl.BlockSpec((nB, Ls), lambda c: (0, 0)),
            pl.BlockSpec((nB, Li), lambda c: (0, 0)),
        ],
        out_specs=[
            pl.BlockSpec((chunk, nB, Hh), lambda c: (c, 0, 0)),
            pl.BlockSpec((chunk, nB, Ls), lambda c: (c, 0, 0)),
            pl.BlockSpec((chunk, nB, Li), lambda c: (c, 0, 0)),
        ],
        out_shape=[
            jax.ShapeDtypeStruct((nsteps, nB, Hh), F32),
            jax.ShapeDtypeStruct((nsteps, nB, Ls), F32),
            jax.ShapeDtypeStruct((nsteps, nB, Li), F32),
        ],
        scratch_shapes=[
            pltpu.VMEM((nB, H4), F32),
            pltpu.VMEM((nB, Hh), F32),
            pltpu.VMEM((nB, Hh), F32),
            pltpu.VMEM((chunk * nB, H4), F32),
        ],
    )(x2d, Wx1, b, Wx2, Wh, Wai, Wc3, cst, h0, c0, memT, infT,
      mask_s, mask_i)
    return dec_out, a_std, a_inf


def kernel(src, tgt, lengths, inflection, inflection_lengths, src_emb,
           enc_Wx, enc_Wh, enc_b, inf_emb, inf_Wx, inf_Wh, inf_b,
           gh_Wa, gh_Wi, gh_Wg, gh_bg, tgt_emb, dec_Wx, dec_Wh, dec_b,
           dec_Wa, dec_Wi, dec_Wc, dec_bc):
    L, B = src.shape
    T = tgt.shape[0]
    LI = inflection.shape[0]
    D = src_emb.shape[1]
    H = enc_Wh.shape[0]

    src_e = _sc_gather(src_emb, src.reshape(-1)).astype(BF16)
    tgt_e = _sc_gather(tgt_emb, tgt.reshape(-1)).astype(BF16)
    inf_e = _sc_gather(inf_emb, inflection.reshape(-1)).astype(BF16)

    neg = jnp.float32(-1e9)
    mask_s = jnp.where(jnp.arange(L)[None, :] < lengths[:, None], 0.0, neg)
    mask_i = jnp.where(jnp.arange(LI)[None, :] < inflection_lengths[:, None],
                       0.0, neg)
    mask_s = mask_s.astype(F32)
    mask_i = mask_i.astype(F32)

    memT, hT, cT = _lstm_encoder(src_e, enc_Wx.astype(BF16),
                                 enc_Wh.astype(BF16),
                                 enc_b.reshape(1, -1), L, 16, B)
    infT, _, _ = _lstm_encoder(inf_e, inf_Wx.astype(BF16),
                               inf_Wh.astype(BF16),
                               inf_b.reshape(1, -1), LI, LI, B)

    Wc3 = dec_Wc[:3 * H]
    Wc4 = dec_Wc[3 * H:]
    gA_s, gA_i, cst = _gated_head(memT, infT, gh_Wa, gh_Wi, gh_Wg,
                                  gh_bg.reshape(1, -1), Wc4,
                                  dec_bc.reshape(1, -1), mask_s, mask_i)

    Wx1 = dec_Wx[:D].astype(BF16)
    Wx2 = dec_Wx[D:].astype(BF16)
    Wai = jnp.concatenate([dec_Wa, dec_Wi], axis=1).astype(BF16)

    dec_out, a_std, a_inf = _decoder(tgt_e, Wx1, dec_b.reshape(1, -1),
                                     Wx2, dec_Wh.astype(BF16), Wai,
                                     Wc3.astype(BF16), cst, hT, cT,
                                     memT, infT, mask_s, mask_i, T, 4, B)

    return (dec_out[:T - 1], a_std[:T - 1], a_inf[:T - 1], gA_s, gA_i)


# speedup vs baseline: 2.0174x; 2.0174x over previous
import functools

import jax
import jax.numpy as jnp
from jax.experimental import pallas as pl
from jax.experimental.pallas import tpu as pltpu
from jax.experimental.pallas import tpu_sc as plsc

F32 = jnp.float32
BF16 = jnp.bfloat16

_SC_CORES = 2
_SC_SUBCORES = 16
_SC_WORKERS = _SC_CORES * _SC_SUBCORES


def _sc_gather(table, idx):
    n = idx.shape[0]
    d = table.shape[1]
    bpw = n // _SC_WORKERS
    mesh = plsc.VectorSubcoreMesh(core_axis_name="c", subcore_axis_name="s")

    @functools.partial(
        pl.kernel,
        mesh=mesh,
        out_type=jax.ShapeDtypeStruct((n, d), table.dtype),
        scratch_types=[
            pltpu.VMEM((bpw,), jnp.int32),
            pltpu.VMEM((bpw, d), table.dtype),
            pltpu.SemaphoreType.DMA,
        ],
    )
    def gather_kernel(table_hbm, idx_hbm, out_hbm, idx_v, rows_v, sem):
        wid = jax.lax.axis_index("s") * _SC_CORES + jax.lax.axis_index("c")
        base = wid * bpw
        pltpu.sync_copy(idx_hbm.at[pl.ds(base, bpw)], idx_v)
        pltpu.async_copy(table_hbm.at[idx_v], rows_v, sem).wait()
        pltpu.sync_copy(rows_v, out_hbm.at[pl.ds(base, bpw)])

    return gather_kernel(table, idx)


def _sigmoid(x):
    return jax.nn.sigmoid(x)


def _vattn(memT, q, mask):
    prod = memT * q.astype(BF16)[:, None, :]
    s = jnp.sum(prod, axis=-1, dtype=F32) + mask
    s = s - jnp.max(s, axis=-1, keepdims=True)
    e = jnp.exp(s)
    a = e / jnp.sum(e, axis=-1, keepdims=True)
    ctx = jnp.sum(a.astype(BF16)[:, :, None] * memT, axis=1, dtype=F32)
    return a, ctx


def _enc_kernel(x_ref, wx_ref, wh_ref, b_ref, memT_ref, hT_ref, cT_ref,
                h_s, c_s, xg_s, *, chunk, nB, Hh):
    ci = pl.program_id(0)

    @pl.when(ci == 0)
    def _():
        h_s[:] = jnp.zeros_like(h_s)
        c_s[:] = jnp.zeros_like(c_s)

    xg_s[:] = jnp.dot(x_ref[:], wx_ref[:], preferred_element_type=F32) + b_ref[:]

    def step(t, _):
        hb = h_s[:].astype(BF16)
        xg = xg_s[pl.ds(t * nB, nB)]
        g3 = xg[:, :3 * Hh] + jnp.dot(hb, wh_ref[:, :3 * Hh],
                                      preferred_element_type=F32)
        i_ = _sigmoid(g3[:, :Hh])
        f_ = _sigmoid(g3[:, Hh:2 * Hh])
        gg = jnp.tanh(g3[:, 2 * Hh:])
        go = xg[:, 3 * Hh:] + jnp.dot(hb, wh_ref[:, 3 * Hh:],
                                      preferred_element_type=F32)
        c2 = f_ * c_s[:] + i_ * gg
        h2 = _sigmoid(go) * jnp.tanh(c2)
        memT_ref[:, t, :] = h2.astype(BF16)
        h_s[:] = h2
        c_s[:] = c2
        return 0

    jax.lax.fori_loop(0, chunk, step, 0, unroll=True)
    hT_ref[:] = h_s[:]
    cT_ref[:] = c_s[:]


def _lstm_encoder(x2d, Wx, Wh, b, nsteps, chunk, nB):
    D = x2d.shape[1]
    Hh = Wh.shape[0]
    H4 = Wx.shape[1]
    nchunks = nsteps // chunk
    kern = functools.partial(_enc_kernel, chunk=chunk, nB=nB, Hh=Hh)
    memT, hT, cT = pl.pallas_call(
        kern,
        grid=(nchunks,),
        in_specs=[
            pl.BlockSpec((chunk * nB, D), lambda c: (c, 0)),
            pl.BlockSpec((D, H4), lambda c: (0, 0)),
            pl.BlockSpec((Hh, H4), lambda c: (0, 0)),
            pl.BlockSpec((1, H4), lambda c: (0, 0)),
        ],
        out_specs=[
            pl.BlockSpec((nB, chunk, Hh), lambda c: (0, c, 0)),
            pl.BlockSpec((nB, Hh), lambda c: (0, 0)),
            pl.BlockSpec((nB, Hh), lambda c: (0, 0)),
        ],
        out_shape=[
            jax.ShapeDtypeStruct((nB, nsteps, Hh), BF16),
            jax.ShapeDtypeStruct((nB, Hh), F32),
            jax.ShapeDtypeStruct((nB, Hh), F32),
        ],
        scratch_shapes=[
            pltpu.VMEM((nB, Hh), F32),
            pltpu.VMEM((nB, Hh), F32),
            pltpu.VMEM((chunk * nB, H4), F32),
        ],
    )(x2d, Wx, Wh, b)
    return memT, hT, cT


def _head_kernel(memT_ref, infT_ref, wa_ref, wi_ref, wg_ref, bg_ref,
                 wc4_ref, bc_ref, ms_ref, mi_ref,
                 gas_ref, gai_ref, cst_ref, *, Hh):
    pos = infT_ref[:, 0, :].astype(F32)
    qs = jnp.dot(pos, wa_ref[:], preferred_element_type=F32)
    qi = jnp.dot(pos, wi_ref[:], preferred_element_type=F32)

    a_s, ctx_s = _vattn(memT_ref[:], qs, ms_ref[:])
    a_i, ctx_i = _vattn(infT_ref[:], qi, mi_ref[:])

    cat = jnp.concatenate([ctx_s, ctx_i], axis=1)
    gate = _sigmoid(jnp.dot(cat, wg_ref[:], preferred_element_type=F32)
                    + bg_ref[:])
    g_mem = gate * ctx_s + (1.0 - gate) * ctx_i

    gas_ref[:] = a_s
    gai_ref[:] = a_i
    cst_ref[:] = jnp.dot(g_mem, wc4_ref[:], preferred_element_type=F32) + bc_ref[:]


def _gated_head(memT, infT, Wa, Wi, Wg, bg, Wc4, bc, mask_s, mask_i):
    nB, Ls, Hh = memT.shape
    Li = infT.shape[1]
    kern = functools.partial(_head_kernel, Hh=Hh)
    gas, gai, cst = pl.pallas_call(
        kern,
        out_shape=[
            jax.ShapeDtypeStruct((nB, Ls), F32),
            jax.ShapeDtypeStruct((nB, Li), F32),
            jax.ShapeDtypeStruct((nB, Hh), F32),
        ],
    )(memT, infT, Wa, Wi, Wg, bg, Wc4, bc, mask_s, mask_i)
    return gas, gai, cst


def _vsoftmax(s):
    s = s - jnp.max(s, axis=-1, keepdims=True)
    e = jnp.exp(s)
    return e / jnp.sum(e, axis=-1, keepdims=True)


def _dec_kernel(xg_ref, wx2_ref, wh_ref, wc1_ref,
                cst_ref, h0_ref, c0_ref, memA_ref, memC_ref,
                infA_ref, infC_ref, ms_ref, mi_ref,
                out_ref, as_ref, ai_ref,
                hwh_s, c_s, fd_s, *, chunk, nB, Hh):
    ci = pl.program_id(0)

    @pl.when(ci == 0)
    def _():
        hwh_s[:] = jnp.dot(h0_ref[:].astype(BF16), wh_ref[:],
                           preferred_element_type=F32)
        c_s[:] = c0_ref[:]
        fd_s[:] = jnp.zeros_like(fd_s)

    memA = memA_ref[:]
    memC = memC_ref[:]
    infA = infA_ref[:]
    infC = infC_ref[:]
    ms = ms_ref[:]
    mi = mi_ref[:]
    cst = cst_ref[:]

    def step(t, _):
        g = xg_ref[pl.ds(t * nB, nB)] + hwh_s[:] + jnp.dot(
            fd_s[:].astype(BF16), wx2_ref[:], preferred_element_type=F32)
        i_ = _sigmoid(g[:, :Hh])
        f_ = _sigmoid(g[:, Hh:2 * Hh])
        gg = jnp.tanh(g[:, 2 * Hh:3 * Hh])
        o_ = _sigmoid(g[:, 3 * Hh:])
        c2 = f_ * c_s[:] + i_ * gg
        h2 = o_ * jnp.tanh(c2)
        h2b = h2.astype(BF16)

        hc1 = jnp.dot(h2b, wc1_ref[:], preferred_element_type=F32)

        hwh_s[:] = jnp.dot(h2b, wh_ref[:], preferred_element_type=F32)

        a_s = _vsoftmax(jnp.sum(memA * h2b[:, None, :], axis=-1,
                                dtype=F32) + ms)
        a_i = _vsoftmax(jnp.sum(infA * h2b[:, None, :], axis=-1,
                                dtype=F32) + mi)
        csC = jnp.sum(a_s.astype(BF16)[:, :, None] * memC, axis=1, dtype=F32)
        ciC = jnp.sum(a_i.astype(BF16)[:, :, None] * infC, axis=1, dtype=F32)

        out = jnp.tanh(hc1 + csC + ciC + cst)

        out_ref[t] = out
        as_ref[t] = a_s
        ai_ref[t] = a_i
        c_s[:] = c2
        fd_s[:] = out
        return 0

    jax.lax.fori_loop(0, chunk, step, 0, unroll=True)


def _decoder(xg, Wx2, Wh, Wc1, cst, h0, c0, memA, memC, infA, infC,
             mask_s, mask_i, nsteps, chunk, nB):
    Hh = h0.shape[1]
    H4 = Wx2.shape[1]
    Ls = memA.shape[1]
    Li = infA.shape[1]
    nchunks = nsteps // chunk
    kern = functools.partial(_dec_kernel, chunk=chunk, nB=nB, Hh=Hh)
    dec_out, a_std, a_inf = pl.pallas_call(
        kern,
        grid=(nchunks,),
        in_specs=[
            pl.BlockSpec((chunk * nB, H4), lambda c: (c, 0)),
            pl.BlockSpec((Hh, H4), lambda c: (0, 0)),
            pl.BlockSpec((Hh, H4), lambda c: (0, 0)),
            pl.BlockSpec((Hh, Hh), lambda c: (0, 0)),
            pl.BlockSpec((nB, Hh), lambda c: (0, 0)),
            pl.BlockSpec((nB, Hh), lambda c: (0, 0)),
            pl.BlockSpec((nB, Hh), lambda c: (0, 0)),
            pl.BlockSpec((nB, Ls, Hh), lambda c: (0, 0, 0)),
            pl.BlockSpec((nB, Ls, Hh), lambda c: (0, 0, 0)),
            pl.BlockSpec((nB, Li, Hh), lambda c: (0, 0, 0)),
            pl.BlockSpec((nB, Li, Hh), lambda c: (0, 0, 0)),
            pl.BlockSpec((nB, Ls), lambda c: (0, 0)),
            pl.BlockSpec((nB, Li), lambda c: (0, 0)),
        ],
        out_specs=[
            pl.BlockSpec((chunk, nB, Hh), lambda c: (c, 0, 0)),
            pl.BlockSpec((chunk, nB, Ls), lambda c: (c, 0, 0)),
            pl.BlockSpec((chunk, nB, Li), lambda c: (c, 0, 0)),
        ],
        out_shape=[
            jax.ShapeDtypeStruct((nsteps, nB, Hh), F32),
            jax.ShapeDtypeStruct((nsteps, nB, Ls), F32),
            jax.ShapeDtypeStruct((nsteps, nB, Li), F32),
        ],
        scratch_shapes=[
            pltpu.VMEM((nB, H4), F32),
            pltpu.VMEM((nB, Hh), F32),
            pltpu.VMEM((nB, Hh), F32),
        ],
    )(xg, Wx2, Wh, Wc1, cst, h0, c0, memA, memC, infA, infC,
      mask_s, mask_i)
    return dec_out, a_std, a_inf


def _mm_kernel(x_ref, w_ref, b_ref, o_ref):
    o_ref[:] = jnp.dot(x_ref[:], w_ref[:],
                       preferred_element_type=F32) + b_ref[:]


def _mm(x, W, b, rows_per_chunk):
    n, d = x.shape
    m = W.shape[1]
    return pl.pallas_call(
        _mm_kernel,
        grid=(n // rows_per_chunk,),
        in_specs=[
            pl.BlockSpec((rows_per_chunk, d), lambda c: (c, 0)),
            pl.BlockSpec((d, m), lambda c: (0, 0)),
            pl.BlockSpec((1, m), lambda c: (0, 0)),
        ],
        out_specs=pl.BlockSpec((rows_per_chunk, m), lambda c: (c, 0)),
        out_shape=jax.ShapeDtypeStruct((n, m), F32),
    )(x, W, b)


def _bank2_kernel(x_ref, w1_ref, w2_ref, a_ref, c_ref):
    x = x_ref[:]
    a_ref[:] = jnp.dot(x, w1_ref[:],
                       preferred_element_type=F32).astype(BF16)
    c_ref[:] = jnp.dot(x, w2_ref[:],
                       preferred_element_type=F32).astype(BF16)


def _bank2(x, W1, W2, rows_per_chunk):
    n, d = x.shape
    m = W1.shape[1]
    return pl.pallas_call(
        _bank2_kernel,
        grid=(n // rows_per_chunk,),
        in_specs=[
            pl.BlockSpec((rows_per_chunk, d), lambda c: (c, 0)),
            pl.BlockSpec((d, m), lambda c: (0, 0)),
            pl.BlockSpec((d, m), lambda c: (0, 0)),
        ],
        out_specs=[
            pl.BlockSpec((rows_per_chunk, m), lambda c: (c, 0)),
            pl.BlockSpec((rows_per_chunk, m), lambda c: (c, 0)),
        ],
        out_shape=[
            jax.ShapeDtypeStruct((n, m), BF16),
            jax.ShapeDtypeStruct((n, m), BF16),
        ],
    )(x, W1, W2)


def kernel(src, tgt, lengths, inflection, inflection_lengths, src_emb,
           enc_Wx, enc_Wh, enc_b, inf_emb, inf_Wx, inf_Wh, inf_b,
           gh_Wa, gh_Wi, gh_Wg, gh_bg, tgt_emb, dec_Wx, dec_Wh, dec_b,
           dec_Wa, dec_Wi, dec_Wc, dec_bc):
    L, B = src.shape
    T = tgt.shape[0]
    LI = inflection.shape[0]
    D = src_emb.shape[1]
    H = enc_Wh.shape[0]

    src_e = _sc_gather(src_emb, src.reshape(-1)).astype(BF16)
    tgt_e = _sc_gather(tgt_emb, tgt.reshape(-1)).astype(BF16)
    inf_e = _sc_gather(inf_emb, inflection.reshape(-1)).astype(BF16)

    neg = jnp.float32(-1e9)
    mask_s = jnp.where(jnp.arange(L)[None, :] < lengths[:, None], 0.0, neg)
    mask_i = jnp.where(jnp.arange(LI)[None, :] < inflection_lengths[:, None],
                       0.0, neg)
    mask_s = mask_s.astype(F32)
    mask_i = mask_i.astype(F32)

    memT, hT, cT = _lstm_encoder(src_e, enc_Wx.astype(BF16),
                                 enc_Wh.astype(BF16),
                                 enc_b.reshape(1, -1), L, 16, B)
    infT, _, _ = _lstm_encoder(inf_e, inf_Wx.astype(BF16),
                               inf_Wh.astype(BF16),
                               inf_b.reshape(1, -1), LI, LI, B)

    Wc3 = dec_Wc[:3 * H]
    Wc4 = dec_Wc[3 * H:]
    gA_s, gA_i, cst = _gated_head(memT, infT, gh_Wa, gh_Wi, gh_Wg,
                                  gh_bg.reshape(1, -1), Wc4,
                                  dec_bc.reshape(1, -1), mask_s, mask_i)

    memA2, memC2 = _bank2(memT.reshape(L * B, H), dec_Wa.T.astype(BF16),
                          dec_Wc[H:2 * H].astype(BF16), 512)
    infA2, infC2 = _bank2(infT.reshape(LI * B, H), dec_Wi.T.astype(BF16),
                          dec_Wc[2 * H:3 * H].astype(BF16), LI * B)

    dec_xg = _mm(tgt_e, dec_Wx[:D].astype(BF16), dec_b.reshape(1, -1), 512)

    dec_out, a_std, a_inf = _decoder(dec_xg, dec_Wx[D:].astype(BF16),
                                     dec_Wh.astype(BF16),
                                     dec_Wc[:H].astype(BF16), cst, hT, cT,
                                     memA2.reshape(B, L, H),
                                     memC2.reshape(B, L, H),
                                     infA2.reshape(B, LI, H),
                                     infC2.reshape(B, LI, H),
                                     mask_s, mask_i, T, 4, B)

    return (dec_out[:T - 1], a_std[:T - 1], a_inf[:T - 1], gA_s, gA_i)
